# baseline (speedup 1.0000x reference)
import functools

import jax
import jax.numpy as jnp
from jax import lax
from jax.experimental import pallas as pl
from jax.experimental.pallas import tpu as pltpu
from jax.experimental.pallas import tpu_sc as plsc

_N = 10000
_E = 320000
_DF = 128
_H = 32
_NMP = 5

_NC = 2
_NS = 16
_NW = _NC * _NS
_CHUNK = 128
_EPT = _E // _NW
_NFULL = _EPT // _CHUNK
_TAIL = _EPT - _NFULL * _CHUNK
_NP = 10240
_RPT = _NP // _NS

_mesh = plsc.VectorSubcoreMesh(core_axis_name="c", subcore_axis_name="s")
_sc_params = pltpu.CompilerParams(use_tc_tiling_on_sc=False)


def _zero_shared_rows(zbuf, s_sh, sid):
    def zb(i, carry):
        zbuf[i, 0:16] = jnp.zeros((16,), jnp.float32)
        zbuf[i, 16:32] = jnp.zeros((16,), jnp.float32)
        return carry
    lax.fori_loop(0, _CHUNK, zb, 0)
    for k in range(_RPT // _CHUNK):
        pltpu.sync_copy(zbuf, s_sh.at[pl.ds(sid * _RPT + k * _CHUNK, _CHUNK)])


def _drain_shared(s_sh, out, cid, sid):
    pltpu.sync_copy(
        s_sh.at[pl.ds(sid * _RPT, _RPT)],
        out.at[cid, pl.ds(sid * _RPT, _RPT)],
    )


def _sc_scatter_loop(load, gbuf, ridx, s_sh, sems):
    load(0, 0, sems[0])

    def outer(g, carry):
        for b in range(2):
            j = g * 2 + b
            nxt = j + 1

            @pl.when(nxt < _NFULL)
            def _():
                load(nxt, 1 - b, sems[1 - b])

            load(j, b, sems[b], wait_only=True)
            pltpu.sync_copy(gbuf.at[b],
                            s_sh.at[ridx.at[pl.ds(j * _CHUNK, _CHUNK)]],
                            add=True)
        return carry
    lax.fori_loop(0, _NFULL // 2, outer, 0)


@functools.partial(
    pl.kernel,
    mesh=_mesh,
    out_type=jax.ShapeDtypeStruct((_NC, _NP, _H), jnp.float32),
    scratch_types=[
        pltpu.VMEM((_EPT,), jnp.int32),
        pltpu.VMEM((_EPT,), jnp.int32),
        pltpu.VMEM((2, _CHUNK, _H), jnp.float32),
        pltpu.VMEM((_CHUNK, _H), jnp.float32),
        pltpu.VMEM_SHARED((_NP, _H), jnp.float32),
        pltpu.SemaphoreType.DMA,
        pltpu.SemaphoreType.DMA,
    ],
    compiler_params=_sc_params,
)
def _sc_gather_scatter(hn, send, recv, out, sidx, ridx, gbuf, zbuf, s_sh,
                       sem0, sem1):
    cid = lax.axis_index("c")
    sid = lax.axis_index("s")
    wid = cid * _NS + sid
    base = wid * _EPT
    pltpu.sync_copy(send.at[pl.ds(base, _EPT)], sidx)
    pltpu.sync_copy(recv.at[pl.ds(base, _EPT)], ridx)
    _zero_shared_rows(zbuf, s_sh, sid)
    plsc.subcore_barrier()

    def load(j, b, sem, wait_only=False):
        cp = (pltpu.make_async_copy if wait_only else pltpu.async_copy)(
            hn.at[sidx.at[pl.ds(j * _CHUNK, _CHUNK)]], gbuf.at[b], sem)
        if wait_only:
            cp.wait()

    _sc_scatter_loop(load, gbuf, ridx, s_sh, (sem0, sem1))

    tb = _NFULL * _CHUNK
    pltpu.async_copy(hn.at[sidx.at[pl.ds(tb, _TAIL)]],
                     gbuf.at[0, pl.ds(0, _TAIL)], sem0).wait()
    pltpu.sync_copy(gbuf.at[0, pl.ds(0, _TAIL)],
                    s_sh.at[ridx.at[pl.ds(tb, _TAIL)]], add=True)

    plsc.subcore_barrier()
    _drain_shared(s_sh, out, cid, sid)


@functools.partial(
    pl.kernel,
    mesh=_mesh,
    out_type=jax.ShapeDtypeStruct((_NC, _NP, _H), jnp.float32),
    scratch_types=[
        pltpu.VMEM((_EPT,), jnp.int32),
        pltpu.VMEM((2, _CHUNK, _H), jnp.float32),
        pltpu.VMEM((_CHUNK, _H), jnp.float32),
        pltpu.VMEM_SHARED((_NP, _H), jnp.float32),
        pltpu.SemaphoreType.DMA,
        pltpu.SemaphoreType.DMA,
    ],
    compiler_params=_sc_params,
)
def _sc_segment_sum(vals, recv, out, ridx, gbuf, zbuf, s_sh, sem0, sem1):
    cid = lax.axis_index("c")
    sid = lax.axis_index("s")
    wid = cid * _NS + sid
    base = wid * _EPT
    pltpu.sync_copy(recv.at[pl.ds(base, _EPT)], ridx)
    _zero_shared_rows(zbuf, s_sh, sid)
    plsc.subcore_barrier()

    def load(j, b, sem, wait_only=False):
        cp = (pltpu.make_async_copy if wait_only else pltpu.async_copy)(
            vals.at[pl.ds(base + j * _CHUNK, _CHUNK)], gbuf.at[b], sem)
        if wait_only:
            cp.wait()

    _sc_scatter_loop(load, gbuf, ridx, s_sh, (sem0, sem1))

    tb = _NFULL * _CHUNK
    pltpu.async_copy(vals.at[pl.ds(base + tb, _TAIL)],
                     gbuf.at[0, pl.ds(0, _TAIL)], sem0).wait()
    pltpu.sync_copy(gbuf.at[0, pl.ds(0, _TAIL)],
                    s_sh.at[ridx.at[pl.ds(tb, _TAIL)]], add=True)

    plsc.subcore_barrier()
    _drain_shared(s_sh, out, cid, sid)


def _full(shape):
    return pl.BlockSpec(shape, lambda i: (0,) * len(shape))


def _mlp2_kernel(x_ref, w1_ref, b1_ref, w2_ref, b2_ref, o_ref):
    y = jnp.dot(x_ref[...], w1_ref[...], preferred_element_type=jnp.float32)
    y = jnp.maximum(y + b1_ref[...], 0.0)
    o_ref[...] = jnp.dot(y, w2_ref[...], preferred_element_type=jnp.float32) + b2_ref[...]


def _mlp2(x, w1, b1, w2, b2, rows_per_block, out_rows=None):
    rows, din = x.shape
    dmid = w1.shape[1]
    dout = w2.shape[1]
    out_rows = rows if out_rows is None else out_rows
    grid = out_rows // rows_per_block
    return pl.pallas_call(
        _mlp2_kernel,
        grid=(grid,),
        in_specs=[
            pl.BlockSpec((rows_per_block, din), lambda i: (i, 0)),
            _full((din, dmid)), _full((1, dmid)),
            _full((dmid, dout)), _full((1, dout)),
        ],
        out_specs=pl.BlockSpec((rows_per_block, dout), lambda i: (i, 0)),
        out_shape=jax.ShapeDtypeStruct((out_rows, dout), jnp.float32),
    )(x, w1, b1.reshape(1, -1), w2, b2.reshape(1, -1))


def _update_kernel(hn_ref, sp_ref0, sp_ref1, ae_ref0, ae_ref1,
                   wmt_ref, w0a_ref, w0b_ref, b0_ref, w1_ref, b1_ref,
                   wnode_ref, lns_ref, lnb_ref, o_ref):
    h = hn_ref[...]
    s = sp_ref0[0] + sp_ref1[0]
    agg = (jnp.dot(s, wmt_ref[...], preferred_element_type=jnp.float32)
           + ae_ref0[0] + ae_ref1[0])
    t = (jnp.dot(h, w0a_ref[...], preferred_element_type=jnp.float32)
         + jnp.dot(agg, w0b_ref[...], preferred_element_type=jnp.float32)
         + b0_ref[...])
    t = jnp.maximum(t, 0.0)
    no = jnp.dot(t, w1_ref[...], preferred_element_type=jnp.float32) + b1_ref[...]
    r = jnp.dot(h, wnode_ref[...], preferred_element_type=jnp.float32) + no
    mu = jnp.mean(r, axis=-1, keepdims=True)
    var = jnp.mean((r - mu) * (r - mu), axis=-1, keepdims=True)
    o_ref[...] = (r - mu) * lax.rsqrt(var + 1e-6) * lns_ref[...] + lnb_ref[...]


def _update(hn, s_p, agge_p, wmt, w0a, w0b, b0, w1, b1, wnode, lns, lnb):
    rb = 1024
    grid = _NP // rb
    row = lambda i: (i, 0)
    return pl.pallas_call(
        _update_kernel,
        grid=(grid,),
        in_specs=[
            pl.BlockSpec((rb, _H), row),
            pl.BlockSpec((1, rb, _H), lambda i: (0, i, 0)),
            pl.BlockSpec((1, rb, _H), lambda i: (1, i, 0)),
            pl.BlockSpec((1, rb, _H), lambda i: (0, i, 0)),
            pl.BlockSpec((1, rb, _H), lambda i: (1, i, 0)),
            _full((_H, _H)), _full((_H, _H)), _full((_H, _H)), _full((1, _H)),
            _full((_H, _H)), _full((1, _H)), _full((_H, _H)),
            _full((1, _H)), _full((1, _H)),
        ],
        out_specs=pl.BlockSpec((rb, _H), row),
        out_shape=jax.ShapeDtypeStruct((_NP, _H), jnp.float32),
    )(hn, s_p, s_p, agge_p, agge_p, wmt, w0a, w0b, b0.reshape(1, -1), w1,
      b1.reshape(1, -1), wnode, lns.reshape(1, -1), lnb.reshape(1, -1))


def kernel(nodes, edges, senders, receivers,
           enc_node_W0, enc_node_b0, enc_node_W1, enc_node_b1,
           enc_edge_W0, enc_edge_b0, enc_edge_W1, enc_edge_b1,
           W_message, W_node,
           nodeMLP_W0, nodeMLP_b0, nodeMLP_W1, nodeMLP_b1,
           ln_scale, ln_bias,
           dec_W0, dec_b0, dec_W1, dec_b1):
    senders = senders.astype(jnp.int32)
    receivers = receivers.astype(jnp.int32)
    nodes_p = jnp.pad(nodes, ((0, _NP - _N), (0, 0)))

    wm_top = W_message[:_H]
    wm_bot = W_message[_H:]
    w2c = enc_edge_W1 @ wm_bot
    b2c = enc_edge_b1 @ wm_bot

    h_n = _mlp2(nodes_p, enc_node_W0, enc_node_b0, enc_node_W1, enc_node_b1, 1024)
    z_e = _mlp2(edges, enc_edge_W0, enc_edge_b0, w2c, b2c, 4000)

    agge_p = _sc_segment_sum(z_e, receivers)

    w0a = nodeMLP_W0[:_H]
    w0b = nodeMLP_W0[_H:]

    for _ in range(_NMP):
        s_p = _sc_gather_scatter(h_n, senders, receivers)
        h_n = _update(h_n, s_p, agge_p,
                      wm_top, w0a, w0b, nodeMLP_b0, nodeMLP_W1, nodeMLP_b1,
                      W_node, ln_scale, ln_bias)

    return _mlp2(h_n, dec_W0, dec_b0, dec_W1, dec_b1, 1000, out_rows=_N)


# speedup vs baseline: 1.4024x; 1.4024x over previous
import functools

import jax
import jax.numpy as jnp
from jax import lax
from jax.experimental import pallas as pl
from jax.experimental.pallas import tpu as pltpu
from jax.experimental.pallas import tpu_sc as plsc

_N = 10000
_E = 320000
_DF = 128
_H = 32
_NMP = 5

_NC = 2
_NS = 16
_NW = _NC * _NS
_CHUNK = 128
_EPT = _E // _NW
_NFULL = _EPT // _CHUNK
_TAIL = _EPT - _NFULL * _CHUNK
_NP = 10240
_RPT = _NP // _NS
_NBUF = 4
_NGRP = 19

_mesh = plsc.VectorSubcoreMesh(core_axis_name="c", subcore_axis_name="s")
_sc_params = pltpu.CompilerParams(use_tc_tiling_on_sc=False)


def _zero_shared_rows(zbuf, s_sh, sid):
    def zb(i, carry):
        zbuf[i, 0:16] = jnp.zeros((16,), jnp.float32)
        zbuf[i, 16:32] = jnp.zeros((16,), jnp.float32)
        return carry
    lax.fori_loop(0, _CHUNK, zb, 0)
    for k in range(_RPT // _CHUNK):
        pltpu.sync_copy(zbuf, s_sh.at[pl.ds(sid * _RPT + k * _CHUNK, _CHUNK)])


def _sc_body(src_of, ridx, gbuf, s_sh, gsems, ssems):
    def sdst(j):
        return s_sh.at[ridx.at[pl.ds(j * _CHUNK, _CHUNK)]]

    for b in range(_NBUF):
        pltpu.async_copy(src_of(b), gbuf.at[b], gsems[b])

    def outer(g, carry):
        for b in range(_NBUF):
            j = g * _NBUF + b
            pltpu.make_async_copy(src_of(j), gbuf.at[b], gsems[b]).wait()
            pltpu.async_copy(gbuf.at[b], sdst(j), ssems[b], add=True)
            nj = j + _NBUF

            @pl.when(nj < _NFULL)
            def _():
                pltpu.make_async_copy(gbuf.at[b], sdst(j), ssems[b]).wait()
                pltpu.async_copy(src_of(nj), gbuf.at[b], gsems[b])
        return carry
    lax.fori_loop(0, _NGRP, outer, 0)

    for b, j in ((0, _NGRP * _NBUF), (1, _NGRP * _NBUF + 1)):
        pltpu.make_async_copy(src_of(j), gbuf.at[b], gsems[b]).wait()
        pltpu.async_copy(gbuf.at[b], sdst(j), ssems[b], add=True)
        pltpu.make_async_copy(gbuf.at[b], sdst(j), ssems[b]).wait()


def _sc_tail(src_tail, ridx, gbuf, s_sh, sem):
    tb = _NFULL * _CHUNK
    pltpu.async_copy(src_tail, gbuf.at[2, pl.ds(0, _TAIL)], sem).wait()
    pltpu.sync_copy(gbuf.at[2, pl.ds(0, _TAIL)],
                    s_sh.at[ridx.at[pl.ds(tb, _TAIL)]], add=True)


def _drain_shared(s_sh, out, cid, sid):
    pltpu.sync_copy(
        s_sh.at[pl.ds(sid * _RPT, _RPT)],
        out.at[cid, pl.ds(sid * _RPT, _RPT)],
    )


_sc_scratch = [
    pltpu.VMEM((_EPT,), jnp.int32),
    pltpu.VMEM((_NBUF, _CHUNK, _H), jnp.float32),
    pltpu.VMEM((_CHUNK, _H), jnp.float32),
    pltpu.VMEM_SHARED((_NP, _H), jnp.float32),
] + [pltpu.SemaphoreType.DMA] * (2 * _NBUF)


@functools.partial(
    pl.kernel,
    mesh=_mesh,
    out_type=jax.ShapeDtypeStruct((_NC, _NP, _H), jnp.float32),
    scratch_types=[pltpu.VMEM((_EPT,), jnp.int32)] + _sc_scratch,
    compiler_params=_sc_params,
)
def _sc_gather_scatter(hn, send, recv, out, sidx, ridx, gbuf, zbuf, s_sh,
                       *sems):
    cid = lax.axis_index("c")
    sid = lax.axis_index("s")
    wid = cid * _NS + sid
    base = wid * _EPT
    pltpu.sync_copy(send.at[pl.ds(base, _EPT)], sidx)
    pltpu.sync_copy(recv.at[pl.ds(base, _EPT)], ridx)
    _zero_shared_rows(zbuf, s_sh, sid)
    plsc.subcore_barrier()

    def src_of(j):
        return hn.at[sidx.at[pl.ds(j * _CHUNK, _CHUNK)]]

    _sc_body(src_of, ridx, gbuf, s_sh, sems[:_NBUF], sems[_NBUF:])
    _sc_tail(hn.at[sidx.at[pl.ds(_NFULL * _CHUNK, _TAIL)]],
             ridx, gbuf, s_sh, sems[0])

    plsc.subcore_barrier()
    _drain_shared(s_sh, out, cid, sid)


@functools.partial(
    pl.kernel,
    mesh=_mesh,
    out_type=jax.ShapeDtypeStruct((_NC, _NP, _H), jnp.float32),
    scratch_types=_sc_scratch,
    compiler_params=_sc_params,
)
def _sc_segment_sum(vals, recv, out, ridx, gbuf, zbuf, s_sh, *sems):
    cid = lax.axis_index("c")
    sid = lax.axis_index("s")
    wid = cid * _NS + sid
    base = wid * _EPT
    pltpu.sync_copy(recv.at[pl.ds(base, _EPT)], ridx)
    _zero_shared_rows(zbuf, s_sh, sid)
    plsc.subcore_barrier()

    def src_of(j):
        return vals.at[pl.ds(base + j * _CHUNK, _CHUNK)]

    _sc_body(src_of, ridx, gbuf, s_sh, sems[:_NBUF], sems[_NBUF:])
    _sc_tail(vals.at[pl.ds(base + _NFULL * _CHUNK, _TAIL)],
             ridx, gbuf, s_sh, sems[0])

    plsc.subcore_barrier()
    _drain_shared(s_sh, out, cid, sid)


def _full(shape):
    return pl.BlockSpec(shape, lambda i: (0,) * len(shape))


def _mlp2_kernel(x_ref, w1_ref, b1_ref, w2_ref, b2_ref, o_ref):
    y = jnp.dot(x_ref[...], w1_ref[...], preferred_element_type=jnp.float32)
    y = jnp.maximum(y + b1_ref[...], 0.0)
    o_ref[...] = jnp.dot(y, w2_ref[...], preferred_element_type=jnp.float32) + b2_ref[...]


def _mlp2(x, w1, b1, w2, b2, rows_per_block, out_rows=None):
    rows, din = x.shape
    dmid = w1.shape[1]
    dout = w2.shape[1]
    out_rows = rows if out_rows is None else out_rows
    grid = out_rows // rows_per_block
    return pl.pallas_call(
        _mlp2_kernel,
        grid=(grid,),
        in_specs=[
            pl.BlockSpec((rows_per_block, din), lambda i: (i, 0)),
            _full((din, dmid)), _full((1, dmid)),
            _full((dmid, dout)), _full((1, dout)),
        ],
        out_specs=pl.BlockSpec((rows_per_block, dout), lambda i: (i, 0)),
        out_shape=jax.ShapeDtypeStruct((out_rows, dout), jnp.float32),
    )(x, w1, b1.reshape(1, -1), w2, b2.reshape(1, -1))


def _edge_enc_kernel(c0_ref, c1_ref, c2_ref, c3_ref, k0_ref, k1_ref, k2_ref,
                     k3_ref, b1_ref, w2_ref, b2_ref, o_ref):
    y = (jnp.dot(c0_ref[...], k0_ref[...], preferred_element_type=jnp.float32)
         + jnp.dot(c1_ref[...], k1_ref[...], preferred_element_type=jnp.float32)
         + jnp.dot(c2_ref[...], k2_ref[...], preferred_element_type=jnp.float32)
         + jnp.dot(c3_ref[...], k3_ref[...], preferred_element_type=jnp.float32))
    y = jnp.maximum(y + b1_ref[...], 0.0)
    for m in range(32):
        z = jnp.dot(y[:, 64 * m:64 * (m + 1)], w2_ref[...],
                    preferred_element_type=jnp.float32) + b2_ref[...]
        o_ref[:, 32 * m:32 * (m + 1)] = z


def _edge_encode(cols, we0, be0, w2c, b2c):
    f32 = jnp.float32
    eye32 = jnp.eye(32, dtype=f32)
    ks = [jnp.kron(eye32, we0[k].reshape(1, -1)) for k in range(4)]
    b1k = jnp.tile(be0, 32)
    rb = 1000
    out = pl.pallas_call(
        _edge_enc_kernel,
        grid=(_N // rb,),
        in_specs=[pl.BlockSpec((rb, _H), lambda i: (i, 0))] * 4
        + [_full((_H, 2048))] * 4
        + [_full((1, 2048)), _full((64, _H)), _full((1, _H))],
        out_specs=pl.BlockSpec((rb, 1024), lambda i: (i, 0)),
        out_shape=jax.ShapeDtypeStruct((_N, 1024), jnp.float32),
    )(*cols, *ks, b1k.reshape(1, -1), w2c, b2c.reshape(1, -1))
    return out.reshape(_E, _H)


def _update_kernel(hn_ref, sp_ref0, sp_ref1, ae_ref0, ae_ref1,
                   wmt_ref, w0a_ref, w0b_ref, b0_ref, w1_ref, b1_ref,
                   wnode_ref, lns_ref, lnb_ref, o_ref):
    h = hn_ref[...]
    s = sp_ref0[0] + sp_ref1[0]
    agg = (jnp.dot(s, wmt_ref[...], preferred_element_type=jnp.float32)
           + ae_ref0[0] + ae_ref1[0])
    t = (jnp.dot(h, w0a_ref[...], preferred_element_type=jnp.float32)
         + jnp.dot(agg, w0b_ref[...], preferred_element_type=jnp.float32)
         + b0_ref[...])
    t = jnp.maximum(t, 0.0)
    no = jnp.dot(t, w1_ref[...], preferred_element_type=jnp.float32) + b1_ref[...]
    r = jnp.dot(h, wnode_ref[...], preferred_element_type=jnp.float32) + no
    mu = jnp.mean(r, axis=-1, keepdims=True)
    var = jnp.mean((r - mu) * (r - mu), axis=-1, keepdims=True)
    o_ref[...] = (r - mu) * lax.rsqrt(var + 1e-6) * lns_ref[...] + lnb_ref[...]


def _update(hn, s_p, agge_p, wmt, w0a, w0b, b0, w1, b1, wnode, lns, lnb):
    rb = 2048
    grid = _NP // rb
    row = lambda i: (i, 0)
    return pl.pallas_call(
        _update_kernel,
        grid=(grid,),
        in_specs=[
            pl.BlockSpec((rb, _H), row),
            pl.BlockSpec((1, rb, _H), lambda i: (0, i, 0)),
            pl.BlockSpec((1, rb, _H), lambda i: (1, i, 0)),
            pl.BlockSpec((1, rb, _H), lambda i: (0, i, 0)),
            pl.BlockSpec((1, rb, _H), lambda i: (1, i, 0)),
            _full((_H, _H)), _full((_H, _H)), _full((_H, _H)), _full((1, _H)),
            _full((_H, _H)), _full((1, _H)), _full((_H, _H)),
            _full((1, _H)), _full((1, _H)),
        ],
        out_specs=pl.BlockSpec((rb, _H), row),
        out_shape=jax.ShapeDtypeStruct((_NP, _H), jnp.float32),
    )(hn, s_p, s_p, agge_p, agge_p, wmt, w0a, w0b, b0.reshape(1, -1), w1,
      b1.reshape(1, -1), wnode, lns.reshape(1, -1), lnb.reshape(1, -1))


def kernel(nodes, edges, senders, receivers,
           enc_node_W0, enc_node_b0, enc_node_W1, enc_node_b1,
           enc_edge_W0, enc_edge_b0, enc_edge_W1, enc_edge_b1,
           W_message, W_node,
           nodeMLP_W0, nodeMLP_b0, nodeMLP_W1, nodeMLP_b1,
           ln_scale, ln_bias,
           dec_W0, dec_b0, dec_W1, dec_b1):
    senders = senders.astype(jnp.int32)
    receivers = receivers.astype(jnp.int32)
    nodes_p = jnp.pad(nodes, ((0, _NP - _N), (0, 0)))

    wm_top = W_message[:_H]
    wm_bot = W_message[_H:]
    w2c = enc_edge_W1 @ wm_bot
    b2c = enc_edge_b1 @ wm_bot

    h_n = _mlp2(nodes_p, enc_node_W0, enc_node_b0, enc_node_W1, enc_node_b1, 1024)
    s_p = _sc_gather_scatter(h_n, senders, receivers)

    cols = [edges[:, k].reshape(_N, _H) for k in range(4)]
    z_e = _edge_encode(cols, enc_edge_W0, enc_edge_b0, w2c, b2c)
    agge_p = _sc_segment_sum(z_e, receivers)

    w0a = nodeMLP_W0[:_H]
    w0b = nodeMLP_W0[_H:]

    for step in range(_NMP):
        h_n = _update(h_n, s_p, agge_p,
                      wm_top, w0a, w0b, nodeMLP_b0, nodeMLP_W1, nodeMLP_b1,
                      W_node, ln_scale, ln_bias)
        if step < _NMP - 1:
            s_p = _sc_gather_scatter(h_n, senders, receivers)

    return _mlp2(h_n, dec_W0, dec_b0, dec_W1, dec_b1, 1000, out_rows=_N)


# speedup vs baseline: 1.4435x; 1.0293x over previous
import functools

import jax
import jax.numpy as jnp
from jax import lax
from jax.experimental import pallas as pl
from jax.experimental.pallas import tpu as pltpu
from jax.experimental.pallas import tpu_sc as plsc

_N = 10000
_E = 320000
_DF = 128
_H = 32
_NMP = 5

_NC = 2
_NS = 16
_NW = _NC * _NS
_CHUNK = 128
_EPT = _E // _NW
_NFULL = _EPT // _CHUNK
_TAIL = _EPT - _NFULL * _CHUNK
_NP = 10240
_RPT = _NP // _NS
_NBUF = 4
_NGRP = 19

_mesh = plsc.VectorSubcoreMesh(core_axis_name="c", subcore_axis_name="s")
_sc_params = pltpu.CompilerParams(use_tc_tiling_on_sc=False)


def _zero_shared_rows(zbuf, s_sh, sid):
    def zb(i, carry):
        zbuf[i, 0:16] = jnp.zeros((16,), jnp.float32)
        zbuf[i, 16:32] = jnp.zeros((16,), jnp.float32)
        return carry
    lax.fori_loop(0, _CHUNK, zb, 0)
    for k in range(_RPT // _CHUNK):
        pltpu.sync_copy(zbuf, s_sh.at[pl.ds(sid * _RPT + k * _CHUNK, _CHUNK)])


def _sc_body(src_of, ridx, gbuf, s_sh, gsems, ssems):
    def sdst(j):
        return s_sh.at[ridx.at[pl.ds(j * _CHUNK, _CHUNK)]]

    for b in range(_NBUF):
        pltpu.async_copy(src_of(b), gbuf.at[b], gsems[b])

    def outer(g, carry):
        for b in range(_NBUF):
            j = g * _NBUF + b
            pltpu.make_async_copy(src_of(j), gbuf.at[b], gsems[b]).wait()
            pltpu.async_copy(gbuf.at[b], sdst(j), ssems[b], add=True)
            nj = j + _NBUF

            @pl.when(nj < _NFULL)
            def _():
                pltpu.make_async_copy(gbuf.at[b], sdst(j), ssems[b]).wait()
                pltpu.async_copy(src_of(nj), gbuf.at[b], gsems[b])
        return carry
    lax.fori_loop(0, _NGRP, outer, 0)

    for b, j in ((0, _NGRP * _NBUF), (1, _NGRP * _NBUF + 1)):
        pltpu.make_async_copy(src_of(j), gbuf.at[b], gsems[b]).wait()
        pltpu.async_copy(gbuf.at[b], sdst(j), ssems[b], add=True)
        pltpu.make_async_copy(gbuf.at[b], sdst(j), ssems[b]).wait()
    for b, j in ((2, _NGRP * _NBUF - 2), (3, _NGRP * _NBUF - 1)):
        pltpu.make_async_copy(gbuf.at[b], sdst(j), ssems[b]).wait()


def _sc_tail(src_tail, ridx, gbuf, s_sh, sem):
    tb = _NFULL * _CHUNK
    pltpu.async_copy(src_tail, gbuf.at[2, pl.ds(0, _TAIL)], sem).wait()
    pltpu.sync_copy(gbuf.at[2, pl.ds(0, _TAIL)],
                    s_sh.at[ridx.at[pl.ds(tb, _TAIL)]], add=True)


def _drain_shared(s_sh, out, cid, sid):
    pltpu.sync_copy(
        s_sh.at[pl.ds(sid * _RPT, _RPT)],
        out.at[cid, pl.ds(sid * _RPT, _RPT)],
    )


_sc_scratch = [
    pltpu.VMEM((_EPT,), jnp.int32),
    pltpu.VMEM((_NBUF, _CHUNK, _H), jnp.float32),
    pltpu.VMEM((_CHUNK, _H), jnp.float32),
    pltpu.VMEM_SHARED((_NP, _H), jnp.float32),
] + [pltpu.SemaphoreType.DMA] * (2 * _NBUF)


@functools.partial(
    pl.kernel,
    mesh=_mesh,
    out_type=jax.ShapeDtypeStruct((_NC, _NP, _H), jnp.float32),
    scratch_types=[pltpu.VMEM((_EPT,), jnp.int32)] + _sc_scratch,
    compiler_params=_sc_params,
)
def _sc_gather_scatter(hn, send, recv, out, sidx, ridx, gbuf, zbuf, s_sh,
                       *sems):
    cid = lax.axis_index("c")
    sid = lax.axis_index("s")
    wid = cid * _NS + sid
    base = wid * _EPT
    pltpu.sync_copy(send.at[pl.ds(base, _EPT)], sidx)
    pltpu.sync_copy(recv.at[pl.ds(base, _EPT)], ridx)
    _zero_shared_rows(zbuf, s_sh, sid)
    plsc.subcore_barrier()

    def src_of(j):
        return hn.at[sidx.at[pl.ds(j * _CHUNK, _CHUNK)]]

    _sc_body(src_of, ridx, gbuf, s_sh, sems[:_NBUF], sems[_NBUF:])
    _sc_tail(hn.at[sidx.at[pl.ds(_NFULL * _CHUNK, _TAIL)]],
             ridx, gbuf, s_sh, sems[0])

    plsc.subcore_barrier()
    _drain_shared(s_sh, out, cid, sid)


@functools.partial(
    pl.kernel,
    mesh=_mesh,
    out_type=jax.ShapeDtypeStruct((_NC, _NP, _H), jnp.float32),
    scratch_types=_sc_scratch,
    compiler_params=_sc_params,
)
def _sc_segment_sum(vals, recv, out, ridx, gbuf, zbuf, s_sh, *sems):
    cid = lax.axis_index("c")
    sid = lax.axis_index("s")
    wid = cid * _NS + sid
    base = wid * _EPT
    pltpu.sync_copy(recv.at[pl.ds(base, _EPT)], ridx)
    _zero_shared_rows(zbuf, s_sh, sid)
    plsc.subcore_barrier()

    def src_of(j):
        return vals.at[pl.ds(base + j * _CHUNK, _CHUNK)]

    _sc_body(src_of, ridx, gbuf, s_sh, sems[:_NBUF], sems[_NBUF:])
    _sc_tail(vals.at[pl.ds(base + _NFULL * _CHUNK, _TAIL)],
             ridx, gbuf, s_sh, sems[0])

    plsc.subcore_barrier()
    _drain_shared(s_sh, out, cid, sid)


def _full(shape):
    return pl.BlockSpec(shape, lambda i: (0,) * len(shape))


def _mlp2_kernel(x_ref, w1_ref, b1_ref, w2_ref, b2_ref, o_ref):
    y = jnp.dot(x_ref[...], w1_ref[...], preferred_element_type=jnp.float32)
    y = jnp.maximum(y + b1_ref[...], 0.0)
    o_ref[...] = jnp.dot(y, w2_ref[...], preferred_element_type=jnp.float32) + b2_ref[...]


def _mlp2(x, w1, b1, w2, b2, rows_per_block, out_rows=None):
    rows, din = x.shape
    dmid = w1.shape[1]
    dout = w2.shape[1]
    out_rows = rows if out_rows is None else out_rows
    grid = out_rows // rows_per_block
    return pl.pallas_call(
        _mlp2_kernel,
        grid=(grid,),
        in_specs=[
            pl.BlockSpec((rows_per_block, din), lambda i: (i, 0)),
            _full((din, dmid)), _full((1, dmid)),
            _full((dmid, dout)), _full((1, dout)),
        ],
        out_specs=pl.BlockSpec((rows_per_block, dout), lambda i: (i, 0)),
        out_shape=jax.ShapeDtypeStruct((out_rows, dout), jnp.float32),
    )(x, w1, b1.reshape(1, -1), w2, b2.reshape(1, -1))


def _edge_enc_kernel(c0_ref, c1_ref, c2_ref, c3_ref, k0_ref, k1_ref, k2_ref,
                     k3_ref, b1_ref, w2_ref, b2_ref, o_ref):
    y = (jnp.dot(c0_ref[...], k0_ref[...], preferred_element_type=jnp.float32)
         + jnp.dot(c1_ref[...], k1_ref[...], preferred_element_type=jnp.float32)
         + jnp.dot(c2_ref[...], k2_ref[...], preferred_element_type=jnp.float32)
         + jnp.dot(c3_ref[...], k3_ref[...], preferred_element_type=jnp.float32))
    y = jnp.maximum(y + b1_ref[...], 0.0)
    for m in range(32):
        z = jnp.dot(y[:, 64 * m:64 * (m + 1)], w2_ref[...],
                    preferred_element_type=jnp.float32) + b2_ref[...]
        o_ref[:, 32 * m:32 * (m + 1)] = z


def _edge_encode(cols, we0, be0, w2c, b2c):
    f32 = jnp.float32
    eye32 = jnp.eye(32, dtype=f32)
    ks = [jnp.kron(eye32, we0[k].reshape(1, -1)) for k in range(4)]
    b1k = jnp.tile(be0, 32)
    rb = 1000
    out = pl.pallas_call(
        _edge_enc_kernel,
        grid=(_N // rb,),
        in_specs=[pl.BlockSpec((rb, _H), lambda i: (i, 0))] * 4
        + [_full((_H, 2048))] * 4
        + [_full((1, 2048)), _full((64, _H)), _full((1, _H))],
        out_specs=pl.BlockSpec((rb, 1024), lambda i: (i, 0)),
        out_shape=jax.ShapeDtypeStruct((_N, 1024), jnp.float32),
    )(*cols, *ks, b1k.reshape(1, -1), w2c, b2c.reshape(1, -1))
    return out.reshape(_E, _H)


def _update_kernel(hn_ref, sp_ref0, sp_ref1, ae_ref0, ae_ref1,
                   wmt_ref, w0a_ref, w0b_ref, b0_ref, w1_ref, b1_ref,
                   wnode_ref, lns_ref, lnb_ref, o_ref):
    h = hn_ref[...]
    s = sp_ref0[0] + sp_ref1[0]
    agg = (jnp.dot(s, wmt_ref[...], preferred_element_type=jnp.float32)
           + ae_ref0[0] + ae_ref1[0])
    t = (jnp.dot(h, w0a_ref[...], preferred_element_type=jnp.float32)
         + jnp.dot(agg, w0b_ref[...], preferred_element_type=jnp.float32)
         + b0_ref[...])
    t = jnp.maximum(t, 0.0)
    no = jnp.dot(t, w1_ref[...], preferred_element_type=jnp.float32) + b1_ref[...]
    r = jnp.dot(h, wnode_ref[...], preferred_element_type=jnp.float32) + no
    mu = jnp.mean(r, axis=-1, keepdims=True)
    var = jnp.mean((r - mu) * (r - mu), axis=-1, keepdims=True)
    o_ref[...] = (r - mu) * lax.rsqrt(var + 1e-6) * lns_ref[...] + lnb_ref[...]


def _update(hn, s_p, agge_p, wmt, w0a, w0b, b0, w1, b1, wnode, lns, lnb):
    rb = 2048
    grid = _NP // rb
    row = lambda i: (i, 0)
    return pl.pallas_call(
        _update_kernel,
        grid=(grid,),
        in_specs=[
            pl.BlockSpec((rb, _H), row),
            pl.BlockSpec((1, rb, _H), lambda i: (0, i, 0)),
            pl.BlockSpec((1, rb, _H), lambda i: (1, i, 0)),
            pl.BlockSpec((1, rb, _H), lambda i: (0, i, 0)),
            pl.BlockSpec((1, rb, _H), lambda i: (1, i, 0)),
            _full((_H, _H)), _full((_H, _H)), _full((_H, _H)), _full((1, _H)),
            _full((_H, _H)), _full((1, _H)), _full((_H, _H)),
            _full((1, _H)), _full((1, _H)),
        ],
        out_specs=pl.BlockSpec((rb, _H), row),
        out_shape=jax.ShapeDtypeStruct((_NP, _H), jnp.float32),
    )(hn, s_p, s_p, agge_p, agge_p, wmt, w0a, w0b, b0.reshape(1, -1), w1,
      b1.reshape(1, -1), wnode, lns.reshape(1, -1), lnb.reshape(1, -1))


def kernel(nodes, edges, senders, receivers,
           enc_node_W0, enc_node_b0, enc_node_W1, enc_node_b1,
           enc_edge_W0, enc_edge_b0, enc_edge_W1, enc_edge_b1,
           W_message, W_node,
           nodeMLP_W0, nodeMLP_b0, nodeMLP_W1, nodeMLP_b1,
           ln_scale, ln_bias,
           dec_W0, dec_b0, dec_W1, dec_b1):
    senders = senders.astype(jnp.int32)
    receivers = receivers.astype(jnp.int32)
    nodes_p = jnp.pad(nodes, ((0, _NP - _N), (0, 0)))

    wm_top = W_message[:_H]
    wm_bot = W_message[_H:]
    w2c = enc_edge_W1 @ wm_bot
    b2c = enc_edge_b1 @ wm_bot

    h_n = _mlp2(nodes_p, enc_node_W0, enc_node_b0, enc_node_W1, enc_node_b1, 1024)
    s_p = _sc_gather_scatter(h_n, senders, receivers)

    cols = [edges[:, k].reshape(_N, _H) for k in range(4)]
    z_e = _edge_encode(cols, enc_edge_W0, enc_edge_b0, w2c, b2c)
    agge_p = _sc_segment_sum(z_e, receivers)

    w0a = nodeMLP_W0[:_H]
    w0b = nodeMLP_W0[_H:]

    for step in range(_NMP):
        h_n = _update(h_n, s_p, agge_p,
                      wm_top, w0a, w0b, nodeMLP_b0, nodeMLP_W1, nodeMLP_b1,
                      W_node, ln_scale, ln_bias)
        if step < _NMP - 1:
            s_p = _sc_gather_scatter(h_n, senders, receivers)

    return _mlp2(h_n, dec_W0, dec_b0, dec_W1, dec_b1, 1000, out_rows=_N)


# speedup vs baseline: 2.0196x; 1.3991x over previous
import functools

import jax
import jax.numpy as jnp
from jax import lax
from jax.experimental import pallas as pl
from jax.experimental.pallas import tpu as pltpu
from jax.experimental.pallas import tpu_sc as plsc

_N = 10000
_E = 320000
_DF = 128
_H = 32
_NMP = 5

_NC = 2
_NS = 16
_NW = _NC * _NS
_CHUNK = 128
_EPT = _E // _NW
_NFULL = _EPT // _CHUNK
_TAIL = _EPT - _NFULL * _CHUNK
_NP = 10240
_RPT = _NP // _NS
_NBUF = 4
_NGRP = 19

_mesh = plsc.VectorSubcoreMesh(core_axis_name="c", subcore_axis_name="s")
_sc_params = pltpu.CompilerParams(use_tc_tiling_on_sc=False)


def _zero_shared_rows(zbuf, s_sh, sid):
    def zb(i, carry):
        zbuf[i, 0:16] = jnp.zeros((16,), jnp.float32)
        zbuf[i, 16:32] = jnp.zeros((16,), jnp.float32)
        return carry
    lax.fori_loop(0, _CHUNK, zb, 0)
    for k in range(_RPT // _CHUNK):
        pltpu.sync_copy(zbuf, s_sh.at[pl.ds(sid * _RPT + k * _CHUNK, _CHUNK)])


def _sc_body(src_of, ridx, gbuf, s_sh, gsems, ssems):
    def sdst(j):
        return s_sh.at[ridx.at[pl.ds(j * _CHUNK, _CHUNK)]]

    for b in range(_NBUF):
        pltpu.async_copy(src_of(b), gbuf.at[b], gsems[b])

    def outer(g, carry):
        for b in range(_NBUF):
            j = g * _NBUF + b
            pltpu.make_async_copy(src_of(j), gbuf.at[b], gsems[b]).wait()
            pltpu.async_copy(gbuf.at[b], sdst(j), ssems[b], add=True)
            nj = j + _NBUF

            @pl.when(nj < _NFULL)
            def _():
                pltpu.make_async_copy(gbuf.at[b], sdst(j), ssems[b]).wait()
                pltpu.async_copy(src_of(nj), gbuf.at[b], gsems[b])
        return carry
    lax.fori_loop(0, _NGRP, outer, 0)

    for b, j in ((0, _NGRP * _NBUF), (1, _NGRP * _NBUF + 1)):
        pltpu.make_async_copy(src_of(j), gbuf.at[b], gsems[b]).wait()
        pltpu.async_copy(gbuf.at[b], sdst(j), ssems[b], add=True)
        pltpu.make_async_copy(gbuf.at[b], sdst(j), ssems[b]).wait()
    for b, j in ((2, _NGRP * _NBUF - 2), (3, _NGRP * _NBUF - 1)):
        pltpu.make_async_copy(gbuf.at[b], sdst(j), ssems[b]).wait()


def _sc_tail(src_tail, ridx, gbuf, s_sh, sem):
    tb = _NFULL * _CHUNK
    pltpu.async_copy(src_tail, gbuf.at[2, pl.ds(0, _TAIL)], sem).wait()
    pltpu.sync_copy(gbuf.at[2, pl.ds(0, _TAIL)],
                    s_sh.at[ridx.at[pl.ds(tb, _TAIL)]], add=True)


def _drain_shared(s_sh, out, cid, sid):
    pltpu.sync_copy(
        s_sh.at[pl.ds(sid * _RPT, _RPT)],
        out.at[cid, pl.ds(sid * _RPT, _RPT)],
    )


_sc_scratch = [
    pltpu.VMEM((_EPT,), jnp.int32),
    pltpu.VMEM((_NBUF, _CHUNK, _H), jnp.float32),
    pltpu.VMEM((_CHUNK, _H), jnp.float32),
    pltpu.VMEM_SHARED((_NP, _H), jnp.float32),
] + [pltpu.SemaphoreType.DMA] * (2 * _NBUF)


@functools.partial(
    pl.kernel,
    mesh=_mesh,
    out_type=jax.ShapeDtypeStruct((_NC, _NP, _H), jnp.float32),
    scratch_types=[pltpu.VMEM((_EPT,), jnp.int32)] + _sc_scratch,
    compiler_params=_sc_params,
)
def _sc_gather_scatter(hn, send, recv, out, sidx, ridx, gbuf, zbuf, s_sh,
                       *sems):
    cid = lax.axis_index("c")
    sid = lax.axis_index("s")
    wid = cid * _NS + sid
    base = wid * _EPT
    pltpu.sync_copy(send.at[pl.ds(base, _EPT)], sidx)
    pltpu.sync_copy(recv.at[pl.ds(base, _EPT)], ridx)
    _zero_shared_rows(zbuf, s_sh, sid)
    plsc.subcore_barrier()

    def src_of(j):
        return hn.at[sidx.at[pl.ds(j * _CHUNK, _CHUNK)]]

    _sc_body(src_of, ridx, gbuf, s_sh, sems[:_NBUF], sems[_NBUF:])
    _sc_tail(hn.at[sidx.at[pl.ds(_NFULL * _CHUNK, _TAIL)]],
             ridx, gbuf, s_sh, sems[0])

    plsc.subcore_barrier()
    _drain_shared(s_sh, out, cid, sid)


@functools.partial(
    pl.kernel,
    mesh=_mesh,
    out_type=jax.ShapeDtypeStruct((_NC, _NP, _H), jnp.float32),
    scratch_types=_sc_scratch,
    compiler_params=_sc_params,
)
def _sc_segment_sum(vals, recv, out, ridx, gbuf, zbuf, s_sh, *sems):
    cid = lax.axis_index("c")
    sid = lax.axis_index("s")
    wid = cid * _NS + sid
    base = wid * _EPT
    pltpu.sync_copy(recv.at[pl.ds(base, _EPT)], ridx)
    _zero_shared_rows(zbuf, s_sh, sid)
    plsc.subcore_barrier()

    def src_of(j):
        return vals.at[pl.ds(base + j * _CHUNK, _CHUNK)]

    _sc_body(src_of, ridx, gbuf, s_sh, sems[:_NBUF], sems[_NBUF:])
    _sc_tail(vals.at[pl.ds(base + _NFULL * _CHUNK, _TAIL)],
             ridx, gbuf, s_sh, sems[0])

    plsc.subcore_barrier()
    _drain_shared(s_sh, out, cid, sid)


def _full(shape):
    return pl.BlockSpec(shape, lambda i: (0,) * len(shape))


def _mlp2_kernel(x_ref, w1_ref, b1_ref, w2_ref, b2_ref, o_ref):
    y = jnp.dot(x_ref[...], w1_ref[...], preferred_element_type=jnp.float32)
    y = jnp.maximum(y + b1_ref[...], 0.0)
    o_ref[...] = jnp.dot(y, w2_ref[...], preferred_element_type=jnp.float32) + b2_ref[...]


def _mlp2(x, w1, b1, w2, b2, rows_per_block, out_rows=None):
    rows, din = x.shape
    dmid = w1.shape[1]
    dout = w2.shape[1]
    out_rows = rows if out_rows is None else out_rows
    grid = out_rows // rows_per_block
    return pl.pallas_call(
        _mlp2_kernel,
        grid=(grid,),
        in_specs=[
            pl.BlockSpec((rows_per_block, din), lambda i: (i, 0)),
            _full((din, dmid)), _full((1, dmid)),
            _full((dmid, dout)), _full((1, dout)),
        ],
        out_specs=pl.BlockSpec((rows_per_block, dout), lambda i: (i, 0)),
        out_shape=jax.ShapeDtypeStruct((out_rows, dout), jnp.float32),
    )(x, w1, b1.reshape(1, -1), w2, b2.reshape(1, -1))


def _edge_enc_kernel(c0_ref, c1_ref, c2_ref, c3_ref, kcat_ref, b1_ref,
                     w2k_ref, b2_ref, o_ref):
    x = jnp.concatenate(
        [c0_ref[...], c1_ref[...], c2_ref[...], c3_ref[...]], axis=1)
    y = jnp.dot(x, kcat_ref[...], preferred_element_type=jnp.float32)
    y = jnp.maximum(y + b1_ref[...], 0.0)
    for q in range(8):
        z = jnp.dot(y[:, 256 * q:256 * (q + 1)], w2k_ref[...],
                    preferred_element_type=jnp.float32) + b2_ref[...]
        o_ref[:, 128 * q:128 * (q + 1)] = z


def _edge_encode(cols, we0, be0, w2c, b2c):
    f32 = jnp.float32
    eye32 = jnp.eye(32, dtype=f32)
    kcat = jnp.concatenate(
        [jnp.kron(eye32, we0[k].reshape(1, -1)) for k in range(4)])
    b1k = jnp.tile(be0, 32)
    w2k = jnp.kron(jnp.eye(4, dtype=f32), w2c)
    b2k = jnp.tile(b2c, 4)
    rb = 1000
    out = pl.pallas_call(
        _edge_enc_kernel,
        grid=(_N // rb,),
        in_specs=[pl.BlockSpec((rb, _H), lambda i: (i, 0))] * 4
        + [_full((128, 2048)), _full((1, 2048)),
           _full((256, 128)), _full((1, 128))],
        out_specs=pl.BlockSpec((rb, 1024), lambda i: (i, 0)),
        out_shape=jax.ShapeDtypeStruct((_N, 1024), jnp.float32),
    )(*cols, kcat, b1k.reshape(1, -1), w2k, b2k.reshape(1, -1))
    return out.reshape(_E, _H)


_NPP = _NP // 4


def _upd_p_kernel(hp_ref, sp_ref0, sp_ref1, ae_ref0, ae_ref1,
                  wmt_ref, w0a_ref, w0b_ref, b0_ref, w1_ref, b1_ref,
                  wnode_ref, mones_ref, lns_ref, lnb_ref, o_ref):
    h = hp_ref[...]
    s = sp_ref0[0] + sp_ref1[0]
    agg = (jnp.dot(s, wmt_ref[...], preferred_element_type=jnp.float32)
           + ae_ref0[0] + ae_ref1[0])
    t = (jnp.dot(h, w0a_ref[...], preferred_element_type=jnp.float32)
         + jnp.dot(agg, w0b_ref[...], preferred_element_type=jnp.float32)
         + b0_ref[...])
    t = jnp.maximum(t, 0.0)
    no = jnp.dot(t, w1_ref[...], preferred_element_type=jnp.float32) + b1_ref[...]
    r = jnp.dot(h, wnode_ref[...], preferred_element_type=jnp.float32) + no
    mu = jnp.dot(r, mones_ref[...], preferred_element_type=jnp.float32)
    d = r - mu
    var = jnp.dot(d * d, mones_ref[...], preferred_element_type=jnp.float32)
    o_ref[...] = d * lax.rsqrt(var + 1e-6) * lns_ref[...] + lnb_ref[...]


def _update_p(hp, s_p, agge_p, pw):
    row = lambda i: (i, 0)
    return pl.pallas_call(
        _upd_p_kernel,
        grid=(1,),
        in_specs=[
            pl.BlockSpec((_NPP, 128), row),
            pl.BlockSpec((1, _NPP, 128), lambda i: (0, i, 0)),
            pl.BlockSpec((1, _NPP, 128), lambda i: (1, i, 0)),
            pl.BlockSpec((1, _NPP, 128), lambda i: (0, i, 0)),
            pl.BlockSpec((1, _NPP, 128), lambda i: (1, i, 0)),
        ] + [_full((128, 128))] * 3 + [_full((1, 128))]
        + [_full((128, 128)), _full((1, 128))]
        + [_full((128, 128))] * 2 + [_full((1, 128))] * 2,
        out_specs=pl.BlockSpec((_NPP, 128), row),
        out_shape=jax.ShapeDtypeStruct((_NPP, 128), jnp.float32),
    )(hp, s_p, s_p, agge_p, agge_p, *pw)


def _dec_p_kernel(hp_ref, w1_ref, b1_ref, w2_ref, b2_ref, o_ref):
    y = jnp.dot(hp_ref[...], w1_ref[...], preferred_element_type=jnp.float32)
    y = jnp.maximum(y + b1_ref[...], 0.0)
    o_ref[...] = jnp.dot(y, w2_ref[...], preferred_element_type=jnp.float32) + b2_ref[...]


def _decode_p(hp, dec_W0, dec_b0, dec_W1, dec_b1):
    f32 = jnp.float32
    e4 = jnp.eye(4, dtype=f32)
    w1p = jnp.kron(e4, dec_W0)
    b1p = jnp.tile(dec_b0, 4)
    w2p = jnp.kron(e4, dec_W1)
    b2p = jnp.tile(dec_b1, 4)
    rb = 256
    out = pl.pallas_call(
        _dec_p_kernel,
        grid=(_NPP // rb,),
        in_specs=[
            pl.BlockSpec((rb, 128), lambda i: (i, 0)),
            _full((128, 256)), _full((1, 256)),
            _full((256, 512)), _full((1, 512)),
        ],
        out_specs=pl.BlockSpec((rb, 512), lambda i: (i, 0)),
        out_shape=jax.ShapeDtypeStruct((_NPP, 512), jnp.float32),
    )(hp, w1p, b1p.reshape(1, -1), w2p, b2p.reshape(1, -1))
    return out.reshape(_NP, _DF)[:_N]


def kernel(nodes, edges, senders, receivers,
           enc_node_W0, enc_node_b0, enc_node_W1, enc_node_b1,
           enc_edge_W0, enc_edge_b0, enc_edge_W1, enc_edge_b1,
           W_message, W_node,
           nodeMLP_W0, nodeMLP_b0, nodeMLP_W1, nodeMLP_b1,
           ln_scale, ln_bias,
           dec_W0, dec_b0, dec_W1, dec_b1):
    senders = senders.astype(jnp.int32)
    receivers = receivers.astype(jnp.int32)
    nodes_p = jnp.pad(nodes, ((0, _NP - _N), (0, 0)))

    f32 = jnp.float32
    wm_top = W_message[:_H]
    wm_bot = W_message[_H:]
    w2c = enc_edge_W1 @ wm_bot
    b2c = enc_edge_b1 @ wm_bot

    e4 = jnp.eye(4, dtype=f32)
    pk = lambda w: jnp.kron(e4, w)
    t4 = lambda b: jnp.tile(b, 4).reshape(1, -1)
    mones = jnp.kron(e4, jnp.full((_H, _H), 1.0 / _H, dtype=f32))
    pw = (pk(wm_top), pk(nodeMLP_W0[:_H]), pk(nodeMLP_W0[_H:]),
          t4(nodeMLP_b0), pk(nodeMLP_W1), t4(nodeMLP_b1), pk(W_node),
          mones, t4(ln_scale), t4(ln_bias))

    h_n = _mlp2(nodes_p, enc_node_W0, enc_node_b0, enc_node_W1, enc_node_b1, 1024)
    hp = h_n.reshape(_NPP, 128)
    s_p = _sc_gather_scatter(h_n, senders, receivers)

    edges_b, hp = lax.optimization_barrier((edges, hp))
    cols = [edges_b[:, k].reshape(_N, _H) for k in range(4)]
    z_e = _edge_encode(cols, enc_edge_W0, enc_edge_b0, w2c, b2c)
    agge_p = _sc_segment_sum(z_e, receivers)
    aggp_v = agge_p.reshape(_NC, _NPP, 128)

    for step in range(_NMP):
        hp = _update_p(hp, s_p.reshape(_NC, _NPP, 128), aggp_v, pw)
        if step < _NMP - 1:
            s_p = _sc_gather_scatter(hp.reshape(_NP, _H), senders, receivers)

    return _decode_p(hp, dec_W0, dec_b0, dec_W1, dec_b1)
